# baseline (speedup 1.0000x reference)
import functools

import jax
import jax.numpy as jnp
from jax import lax
from jax.experimental import pallas as pl
from jax.experimental.pallas import tpu as pltpu
from jax.experimental.pallas import tpu_sc as plsc

B = 16384
L = 50
D = 32
V = 1000000
LANES = 16

NC, NS = 2, 16
NW = NC * NS
RPB = 2
IPB = RPB * L
NBLK = B // RPB
BPW = NBLK // NW
ROWS_PW = B // NW
RING = 8

_mesh = plsc.VectorSubcoreMesh(
    core_axis_name="c", subcore_axis_name="s", num_cores=NC, num_subcores=NS
)


@functools.partial(
    pl.kernel,
    out_type=jax.ShapeDtypeStruct((B, D), jnp.float32),
    mesh=_mesh,
    scratch_types=[
        pltpu.VMEM((BPW, IPB), jnp.int32),
        pltpu.VMEM((BPW, IPB), jnp.int32),
        pltpu.VMEM((RING, IPB, D), jnp.float32),
        pltpu.VMEM((ROWS_PW, D), jnp.float32),
        pltpu.SemaphoreType.DMA,
    ]
    + [pltpu.SemaphoreType.DMA] * RING,
    compiler_params=pltpu.CompilerParams(use_tc_tiling_on_sc=False),
)
def _sc_sum(idx_hbm, table_hbm, out_hbm, idx_v, idxq_v, rows_v, out_v,
            sem_i, *sems):
    wid = lax.axis_index("s") * NC + lax.axis_index("c")
    blk0 = wid * BPW

    idx_cp = pltpu.make_async_copy(
        idx_hbm.at[pl.ds(blk0, BPW), :], idx_v, sem_i
    )
    idx_cp.start()
    idx_cp.wait()

    def shift_row(r, carry):
        for off in (0, 16, 32, 48, 64, 80, IPB - LANES):
            v = idx_v[r, pl.ds(off, LANES)]
            idxq_v[r, pl.ds(off, LANES)] = (
                ((v >> 9) << 9) + ((v & 127) << 2) + ((v >> 7) & 3)
            )
        return carry

    lax.fori_loop(0, BPW, shift_row, 0)

    for s in range(RING):
        pltpu.make_async_copy(
            table_hbm.at[idxq_v.at[s]], rows_v.at[s], sems[s]
        ).start()

    def body(k, carry):
        for s in range(RING):
            j = k * RING + s
            pltpu.make_async_copy(
                table_hbm.at[idxq_v.at[j]], rows_v.at[s], sems[s]
            ).wait()
            for r in range(RPB):
                base = r * L
                a0 = rows_v[s, base, pl.ds(0, LANES)]
                a1 = rows_v[s, base, pl.ds(LANES, LANES)]
                for q in range(1, L):
                    a0 = a0 + rows_v[s, base + q, pl.ds(0, LANES)]
                    a1 = a1 + rows_v[s, base + q, pl.ds(LANES, LANES)]
                orow = j * RPB + r
                out_v[orow, pl.ds(0, LANES)] = a0
                out_v[orow, pl.ds(LANES, LANES)] = a1
            nxt = j + RING

            @pl.when(nxt < BPW)
            def _():
                pltpu.make_async_copy(
                    table_hbm.at[idxq_v.at[nxt]], rows_v.at[s], sems[s]
                ).start()

        return carry

    lax.fori_loop(0, BPW // RING, body, 0)

    pltpu.sync_copy(out_v, out_hbm.at[pl.ds(wid * ROWS_PW, ROWS_PW), :])


_TP_COLS = 1024
_TP_ROWS = _TP_COLS // 4
_TP_GRID = -(-V // _TP_COLS)


def _tp_body(tt_ref, out_ref):
    for b in range(_TP_COLS // 512):
        x = tt_ref[:, pl.ds(b * 512, 512)]
        for a in range(4):
            out_ref[pl.ds(b * 128, 128), pl.ds(a * D, D)] = (
                x[:, a * 128:(a + 1) * 128].T
            )


_tp = pl.pallas_call(
    _tp_body,
    grid=(_TP_GRID,),
    in_specs=[pl.BlockSpec((D, _TP_COLS), lambda i: (0, i))],
    out_specs=pl.BlockSpec((_TP_ROWS, 128), lambda i: (i, 0)),
    out_shape=jax.ShapeDtypeStruct((_TP_GRID * _TP_ROWS, 128), jnp.float32),
)


def _fin_body(idx_ref, sums_ref, t0_ref, out_ref):
    idx = idx_ref[...]
    sums = sums_ref[...]
    t0 = t0_ref[...]
    cnt = jnp.sum((idx != 0).astype(jnp.float32), axis=1, keepdims=True)
    nz = jnp.float32(L) - cnt
    mean = (sums - nz * t0) / cnt
    nrm = jnp.sqrt(jnp.sum(mean * mean, axis=1, keepdims=True))
    out_ref[...] = mean / jnp.maximum(nrm, 1e-12)


_FIN_BLK = 1024

_fin = pl.pallas_call(
    _fin_body,
    grid=(B // _FIN_BLK,),
    in_specs=[
        pl.BlockSpec((_FIN_BLK, L), lambda i: (i, 0)),
        pl.BlockSpec((_FIN_BLK, D), lambda i: (i, 0)),
        pl.BlockSpec((1, D), lambda i: (0, 0)),
    ],
    out_specs=pl.BlockSpec((_FIN_BLK, D), lambda i: (i, 0)),
    out_shape=jax.ShapeDtypeStruct((B, D), jnp.float32),
)


@jax.jit
def kernel(ingredient_indices, table):
    idx2d = ingredient_indices.reshape(NBLK, IPB)
    tt = table.T
    t32 = _tp(tt).reshape(4 * _TP_GRID * _TP_ROWS, D)
    sums = _sc_sum(idx2d, t32)
    return _fin(ingredient_indices, sums, table[0:1])


# speedup vs baseline: 1.4237x; 1.4237x over previous
import functools

import jax
import jax.numpy as jnp
from jax import lax
from jax.experimental import pallas as pl
from jax.experimental.pallas import tpu as pltpu
from jax.experimental.pallas import tpu_sc as plsc

B = 16384
L = 50
D = 32
V = 1000000
LANES = 16

NC, NS = 2, 16
NW = NC * NS
RPB = 2
IPB = RPB * L
NBLK = B // RPB
BPW = NBLK // NW
ROWS_PW = B // NW
RING = 8

_mesh = plsc.VectorSubcoreMesh(
    core_axis_name="c", subcore_axis_name="s", num_cores=NC, num_subcores=NS
)


@functools.partial(
    pl.kernel,
    out_type=jax.ShapeDtypeStruct((B, D), jnp.float32),
    mesh=_mesh,
    scratch_types=[
        pltpu.VMEM((BPW, IPB), jnp.int32),
        pltpu.VMEM((BPW, IPB), jnp.int32),
        pltpu.VMEM((RING, IPB, D), jnp.float32),
        pltpu.VMEM((ROWS_PW, D), jnp.float32),
        pltpu.SemaphoreType.DMA,
    ]
    + [pltpu.SemaphoreType.DMA] * RING,
    compiler_params=pltpu.CompilerParams(use_tc_tiling_on_sc=False),
)
def _sc_sum(idx_hbm, table_hbm, out_hbm, idx_v, idxq_v, rows_v, out_v,
            sem_i, *sems):
    wid = lax.axis_index("s") * NC + lax.axis_index("c")
    blk0 = wid * BPW

    idx_cp = pltpu.make_async_copy(
        idx_hbm.at[pl.ds(blk0, BPW), :], idx_v, sem_i
    )
    idx_cp.start()
    idx_cp.wait()

    def shift_row(r, carry):
        for off in (0, 16, 32, 48, 64, 80, IPB - LANES):
            v = idx_v[r, pl.ds(off, LANES)]
            idxq_v[r, pl.ds(off, LANES)] = (
                ((v >> 9) << 9) + ((v & 127) << 2) + ((v >> 7) & 3)
            )
        return carry

    lax.fori_loop(0, BPW, shift_row, 0)

    for s in range(RING):
        pltpu.make_async_copy(
            table_hbm.at[idxq_v.at[s]], rows_v.at[s], sems[s]
        ).start()

    def body(k, carry):
        for s in range(RING):
            j = k * RING + s
            pltpu.make_async_copy(
                table_hbm.at[idxq_v.at[j]], rows_v.at[s], sems[s]
            ).wait()
            for r in range(RPB):
                base = r * L
                a0 = rows_v[s, base, pl.ds(0, LANES)]
                a1 = rows_v[s, base, pl.ds(LANES, LANES)]
                for q in range(1, L):
                    a0 = a0 + rows_v[s, base + q, pl.ds(0, LANES)]
                    a1 = a1 + rows_v[s, base + q, pl.ds(LANES, LANES)]
                orow = j * RPB + r
                out_v[orow, pl.ds(0, LANES)] = a0
                out_v[orow, pl.ds(LANES, LANES)] = a1
            nxt = j + RING

            @pl.when(nxt < BPW)
            def _():
                pltpu.make_async_copy(
                    table_hbm.at[idxq_v.at[nxt]], rows_v.at[s], sems[s]
                ).start()

        return carry

    lax.fori_loop(0, BPW // RING, body, 0)

    pltpu.sync_copy(out_v, out_hbm.at[pl.ds(wid * ROWS_PW, ROWS_PW), :])


_TP_COLS = 2048
_TP_ROWS = _TP_COLS // 4
_TP_GRID = -(-V // _TP_COLS)


def _tp_body(tt_ref, out_ref):
    for b in range(_TP_COLS // 512):
        x = tt_ref[:, pl.ds(b * 512, 512)]
        for a in range(4):
            out_ref[pl.ds(b * 128, 128), pl.ds(a * D, D)] = (
                x[:, a * 128:(a + 1) * 128].T
            )


_tp = pl.pallas_call(
    _tp_body,
    grid=(_TP_GRID,),
    in_specs=[pl.BlockSpec((D, _TP_COLS), lambda i: (0, i))],
    out_specs=pl.BlockSpec((_TP_ROWS, 128), lambda i: (i, 0)),
    out_shape=jax.ShapeDtypeStruct((_TP_GRID * _TP_ROWS, 128), jnp.float32),
)


def _fin_body(idx_ref, sums_ref, t0_ref, out_ref):
    idx = idx_ref[...]
    sums = sums_ref[...]
    t0 = t0_ref[...]
    cnt = jnp.sum((idx != 0).astype(jnp.float32), axis=1, keepdims=True)
    nz = jnp.float32(L) - cnt
    mean = (sums - nz * t0) / cnt
    nrm = jnp.sqrt(jnp.sum(mean * mean, axis=1, keepdims=True))
    out_ref[...] = mean / jnp.maximum(nrm, 1e-12)


_FIN_BLK = 1024

_fin = pl.pallas_call(
    _fin_body,
    grid=(B // _FIN_BLK,),
    in_specs=[
        pl.BlockSpec((_FIN_BLK, L), lambda i: (i, 0)),
        pl.BlockSpec((_FIN_BLK, D), lambda i: (i, 0)),
        pl.BlockSpec((1, D), lambda i: (0, 0)),
    ],
    out_specs=pl.BlockSpec((_FIN_BLK, D), lambda i: (i, 0)),
    out_shape=jax.ShapeDtypeStruct((B, D), jnp.float32),
)


@jax.jit
def kernel(ingredient_indices, table):
    idx2d = ingredient_indices.reshape(NBLK, IPB)
    tt = table.T
    t32 = _tp(tt).reshape(4 * _TP_GRID * _TP_ROWS, D)
    sums = _sc_sum(idx2d, t32)
    return _fin(ingredient_indices, sums, table[0:1])
